# Initial kernel scaffold; baseline (speedup 1.0000x reference)
import jax
import jax.numpy as jnp
from jax.experimental import pallas as pl


def kernel(coords, values):
    raise NotImplementedError("write your pallas kernel here")


# speedup vs baseline: 21.2282x; 21.2282x over previous
import functools

import jax
import jax.numpy as jnp
from jax import lax
from jax.experimental import pallas as pl
from jax.experimental.pallas import tpu as pltpu
from jax.experimental.pallas import tpu_sc as plsc

N = 262144
B = 4
C = 16
H = 16
NBIN = H * H
NC = 2
NS = 16
NW = NC * NS
NP = N // NW
CH = 512
NCH = NP // CH
GRP = CH // 16


def _sc_body(xs_h, ys_h, eb_h, vals_h, psums_h, pcnt_h,
             xbuf, ybuf, ebv, vbuf, acc, cnt, sem0, sem1):
    cid = lax.axis_index("c")
    sid = lax.axis_index("s")
    wid = sid * NC + cid
    base = wid * NP

    pltpu.sync_copy(xs_h.at[pl.ds(base, NP)], xbuf)
    pltpu.sync_copy(ys_h.at[pl.ds(base, NP)], ybuf)
    pltpu.sync_copy(eb_h, ebv)

    zero16 = jnp.zeros((16,), jnp.float32)
    ones16 = jnp.ones((16,), jnp.float32)
    iota16 = lax.broadcasted_iota(jnp.int32, (16,), 0)

    def zacc(i, carry):
        acc[i, :] = zero16
        return carry
    lax.fori_loop(0, B * NBIN, zacc, 0)

    def zcnt(i, carry):
        cnt[pl.ds(i * 16, 16)] = zero16
        return carry
    lax.fori_loop(0, NBIN // 16, zcnt, 0)

    evs = [ebv[e, :] for e in range(9)]

    sems = (sem0, sem1)

    def copies(slot, c):
        start = base + c * CH
        return [pltpu.make_async_copy(vals_h.at[b, pl.ds(start, CH), :],
                                      vbuf.at[slot, b], sems[slot])
                for b in range(B)]

    for s in range(2):
        for cp in copies(s, jnp.int32(s)):
            cp.start()

    def process(slot, c):
        def grp_body(g, carry):
            j0 = c * CH + g * 16
            X = xbuf[pl.ds(j0, 16)]
            Y = ybuf[pl.ds(j0, 16)]
            sx = jnp.zeros((16,), jnp.int32)
            sy = jnp.zeros((16,), jnp.int32)
            for ev in evs:
                sx = sx + (ev <= X).astype(jnp.int32)
                sy = sy + (ev <= Y).astype(jnp.int32)
            bins = sx + sy * 16 + 119
            plsc.addupdate_scatter(cnt, [bins], ones16)
            rows = g * 16 + iota16
            for b in range(B):
                rb = bins + b * NBIN
                for cc in range(C):
                    col = jnp.full((16,), cc, jnp.int32)
                    v = plsc.load_gather(vbuf.at[slot, b], [rows, col])
                    plsc.addupdate_scatter(acc, [rb, col], v)
            return carry
        lax.fori_loop(0, GRP, grp_body, 0)

    def pair_body(p, carry):
        for s in range(2):
            c = 2 * p + s
            for cp in copies(s, c):
                cp.wait()
            process(s, c)
            @pl.when(c + 2 < NCH)
            def _():
                for cp in copies(s, c + 2):
                    cp.start()
        return carry
    lax.fori_loop(0, NCH // 2, pair_body, 0)

    pltpu.sync_copy(acc, psums_h.at[wid])
    pltpu.sync_copy(cnt, pcnt_h.at[wid])


@functools.partial(
    pl.kernel,
    out_type=(jax.ShapeDtypeStruct((NW, B * NBIN, C), jnp.float32),
              jax.ShapeDtypeStruct((NW, NBIN), jnp.float32)),
    mesh=plsc.VectorSubcoreMesh(core_axis_name="c", subcore_axis_name="s"),
    compiler_params=pltpu.CompilerParams(use_tc_tiling_on_sc=False, needs_layout_passes=False),
    scratch_types=[
        pltpu.VMEM((NP,), jnp.float32),
        pltpu.VMEM((NP,), jnp.float32),
        pltpu.VMEM((9, 16), jnp.float32),
        pltpu.VMEM((2, B, CH, C), jnp.float32),
        pltpu.VMEM((B * NBIN, C), jnp.float32),
        pltpu.VMEM((NBIN,), jnp.float32),
        pltpu.SemaphoreType.DMA,
        pltpu.SemaphoreType.DMA,
    ],
)
def _sc_partials(xs_h, ys_h, eb_h, vals_h, psums_h, pcnt_h, *scratch):
    _sc_body(xs_h, ys_h, eb_h, vals_h, psums_h, pcnt_h, *scratch)


def _combine_body(ps_ref, pc_ref, out_ref):
    sums = jnp.sum(ps_ref[...], axis=0)
    counts = jnp.sum(pc_ref[...], axis=0)
    out_ref[...] = sums / counts[None, :, None]


def kernel(coords, values):
    xs = coords[:, 0]
    ys = coords[:, 1]
    edges = jnp.linspace(-1.0 - 1e-6, 1.0 + 1e-6, H + 1).astype(coords.dtype)
    eb = jnp.broadcast_to(edges[8:17, None], (9, 16)).astype(jnp.float32)
    psums, pcnt = _sc_partials(xs, ys, eb, values)
    means = pl.pallas_call(
        _combine_body,
        out_shape=jax.ShapeDtypeStruct((B, NBIN, C), jnp.float32),
    )(psums.reshape(NW, B, NBIN, C), pcnt)
    return means.reshape(B, NBIN * C)


# speedup vs baseline: 70.6787x; 3.3295x over previous
import functools

import jax
import jax.numpy as jnp
from jax import lax
from jax.experimental import pallas as pl
from jax.experimental.pallas import tpu as pltpu
from jax.experimental.pallas import tpu_sc as plsc

N = 262144
B = 4
C = 16
H = 16
NBIN = H * H
NC = 2
NS = 16
NW = NC * NS
NP = N // NW
CH = 512
NCH = NP // CH
GRP = CH // 16


def _sc_body(ct_h, eb_h, vt_h, psums_h, pcnt_h,
             xbuf, ybuf, ebv, vbuf, acc, cnt, sem0, sem1):
    cid = lax.axis_index("c")
    sid = lax.axis_index("s")
    wid = sid * NC + cid
    base = wid * NP

    pltpu.sync_copy(ct_h.at[0, pl.ds(base, NP)], xbuf)
    pltpu.sync_copy(ct_h.at[1, pl.ds(base, NP)], ybuf)
    pltpu.sync_copy(eb_h, ebv)

    zero16 = jnp.zeros((16,), jnp.float32)
    ones16 = jnp.ones((16,), jnp.float32)

    def zacc(i, carry):
        for k in range(NBIN // 16):
            acc[i, pl.ds(k * 16, 16)] = zero16
        return carry
    lax.fori_loop(0, B * C, zacc, 0)

    def zcnt(i, carry):
        cnt[pl.ds(i * 16, 16)] = zero16
        return carry
    lax.fori_loop(0, NBIN // 16, zcnt, 0)

    evs = [ebv[e, :] for e in range(9)]

    sems = (sem0, sem1)

    def copies(slot, c):
        start = base + c * CH
        return [pltpu.make_async_copy(vt_h.at[b, :, pl.ds(start, CH)],
                                      vbuf.at[slot, b], sems[slot])
                for b in range(B)]

    for s in range(2):
        for cp in copies(s, jnp.int32(s)):
            cp.start()

    rowc = [jnp.full((16,), r, jnp.int32) for r in range(B * C)]

    def process(slot, c):
        def grp_body(g, carry):
            j0 = c * CH + g * 16
            X = xbuf[pl.ds(j0, 16)]
            Y = ybuf[pl.ds(j0, 16)]
            sx = jnp.zeros((16,), jnp.int32)
            sy = jnp.zeros((16,), jnp.int32)
            for ev in evs:
                sx = sx + (ev <= X).astype(jnp.int32)
                sy = sy + (ev <= Y).astype(jnp.int32)
            bins = sx + sy * 16 + 119
            plsc.addupdate_scatter(cnt, [bins], ones16)
            jj = g * 16
            for b in range(B):
                for cc in range(C):
                    v = vbuf[slot, b, cc, pl.ds(jj, 16)]
                    plsc.addupdate_scatter(acc, [rowc[b * C + cc], bins], v)
            return carry
        lax.fori_loop(0, GRP, grp_body, 0)

    def pair_body(p, carry):
        for s in range(2):
            c = 2 * p + s
            for cp in copies(s, c):
                cp.wait()
            process(s, c)
            @pl.when(c + 2 < NCH)
            def _():
                for cp in copies(s, c + 2):
                    cp.start()
        return carry
    lax.fori_loop(0, NCH // 2, pair_body, 0)

    pltpu.sync_copy(acc, psums_h.at[wid])
    pltpu.sync_copy(cnt, pcnt_h.at[wid])


@functools.partial(
    pl.kernel,
    out_type=(jax.ShapeDtypeStruct((NW, B * C, NBIN), jnp.float32),
              jax.ShapeDtypeStruct((NW, NBIN), jnp.float32)),
    mesh=plsc.VectorSubcoreMesh(core_axis_name="c", subcore_axis_name="s"),
    compiler_params=pltpu.CompilerParams(use_tc_tiling_on_sc=False, needs_layout_passes=False),
    scratch_types=[
        pltpu.VMEM((NP,), jnp.float32),
        pltpu.VMEM((NP,), jnp.float32),
        pltpu.VMEM((9, 16), jnp.float32),
        pltpu.VMEM((2, B, C, CH), jnp.float32),
        pltpu.VMEM((B * C, NBIN), jnp.float32),
        pltpu.VMEM((NBIN,), jnp.float32),
        pltpu.SemaphoreType.DMA,
        pltpu.SemaphoreType.DMA,
    ],
)
def _sc_partials(ct_h, eb_h, vt_h, psums_h, pcnt_h, *scratch):
    _sc_body(ct_h, eb_h, vt_h, psums_h, pcnt_h, *scratch)


def _combine_body(ps_ref, pc_ref, out_ref):
    sums = jnp.sum(ps_ref[...], axis=0)
    counts = jnp.sum(pc_ref[...], axis=0)
    out_ref[...] = sums / counts[None, :]


def kernel(coords, values):
    ct = coords.T
    vt = jnp.transpose(values, (0, 2, 1))
    edges = jnp.linspace(-1.0 - 1e-6, 1.0 + 1e-6, H + 1).astype(coords.dtype)
    eb = jnp.broadcast_to(edges[8:17, None], (9, 16)).astype(jnp.float32)
    psums, pcnt = _sc_partials(ct, eb, vt)
    means = pl.pallas_call(
        _combine_body,
        out_shape=jax.ShapeDtypeStruct((B * C, NBIN), jnp.float32),
    )(psums, pcnt)
    return jnp.transpose(means.reshape(B, C, NBIN), (0, 2, 1)).reshape(B, NBIN * C)


# speedup vs baseline: 133.8581x; 1.8939x over previous
import functools

import jax
import jax.numpy as jnp
from jax import lax
from jax.experimental import pallas as pl
from jax.experimental.pallas import tpu as pltpu
from jax.experimental.pallas import tpu_sc as plsc

N = 262144
B = 4
C = 16
H = 16
NBIN = H * H
NC = 2
NS = 16
NW = NC * NS
NP = N // NW
CH = 512
NCH = NP // CH
GRP = CH // 16


def _sc_body(ct_h, eb_h, vt_h, psums_h, pcnt_h,
             xbuf, ybuf, ebv, vbuf, acc, cnt, sem0, sem1):
    cid = lax.axis_index("c")
    sid = lax.axis_index("s")
    wid = sid * NC + cid
    base = wid * NP

    pltpu.sync_copy(ct_h.at[0, pl.ds(base, NP)], xbuf)
    pltpu.sync_copy(ct_h.at[1, pl.ds(base, NP)], ybuf)
    pltpu.sync_copy(eb_h, ebv)

    zero16 = jnp.zeros((16,), jnp.float32)
    ones16 = jnp.ones((16,), jnp.float32)

    def zacc(i, carry):
        for k in range(NBIN // 16):
            acc[i, pl.ds(k * 16, 16)] = zero16
        return carry
    lax.fori_loop(0, B * C, zacc, 0)

    def zcnt(i, carry):
        cnt[pl.ds(i * 16, 16)] = zero16
        return carry
    lax.fori_loop(0, NBIN // 16, zcnt, 0)

    evs = [ebv[e, :] for e in range(9)]

    sems = (sem0, sem1)

    def copies(slot, c):
        start = base + c * CH
        return [pltpu.make_async_copy(vt_h.at[b, :, pl.ds(start, CH)],
                                      vbuf.at[slot, b], sems[slot])
                for b in range(B)]

    for s in range(2):
        for cp in copies(s, jnp.int32(s)):
            cp.start()

    rowc = [jnp.full((16,), r, jnp.int32) for r in range(B * C)]

    def process(slot, c):
        @plsc.parallel_loop(0, GRP)
        def grp_body(g):
            j0 = c * CH + g * 16
            X = xbuf[pl.ds(j0, 16)]
            Y = ybuf[pl.ds(j0, 16)]
            sx = jnp.zeros((16,), jnp.int32)
            sy = jnp.zeros((16,), jnp.int32)
            for ev in evs:
                sx = sx + (ev <= X).astype(jnp.int32)
                sy = sy + (ev <= Y).astype(jnp.int32)
            bins = sx + sy * 16 + 119
            plsc.addupdate_scatter(cnt, [bins], ones16)
            jj = g * 16
            for b in range(B):
                vs = [vbuf[slot, b, cc, pl.ds(jj, 16)] for cc in range(C)]
                for cc in range(C):
                    plsc.addupdate_scatter(acc, [rowc[b * C + cc], bins], vs[cc])

    def pair_body(p, carry):
        for s in range(2):
            c = 2 * p + s
            for cp in copies(s, c):
                cp.wait()
            process(s, c)
            @pl.when(c + 2 < NCH)
            def _():
                for cp in copies(s, c + 2):
                    cp.start()
        return carry
    lax.fori_loop(0, NCH // 2, pair_body, 0)

    pltpu.sync_copy(acc, psums_h.at[wid])
    pltpu.sync_copy(cnt, pcnt_h.at[wid])


@functools.partial(
    pl.kernel,
    out_type=(jax.ShapeDtypeStruct((NW, B * C, NBIN), jnp.float32),
              jax.ShapeDtypeStruct((NW, NBIN), jnp.float32)),
    mesh=plsc.VectorSubcoreMesh(core_axis_name="c", subcore_axis_name="s"),
    compiler_params=pltpu.CompilerParams(use_tc_tiling_on_sc=True, needs_layout_passes=False),
    scratch_types=[
        pltpu.VMEM((NP,), jnp.float32),
        pltpu.VMEM((NP,), jnp.float32),
        pltpu.VMEM((9, 16), jnp.float32),
        pltpu.VMEM((2, B, C, CH), jnp.float32),
        pltpu.VMEM((B * C, NBIN), jnp.float32),
        pltpu.VMEM((NBIN,), jnp.float32),
        pltpu.SemaphoreType.DMA,
        pltpu.SemaphoreType.DMA,
    ],
)
def _sc_partials(ct_h, eb_h, vt_h, psums_h, pcnt_h, *scratch):
    _sc_body(ct_h, eb_h, vt_h, psums_h, pcnt_h, *scratch)


def _combine_body(ps_ref, pc_ref, out_ref):
    sums = jnp.sum(ps_ref[...], axis=0)
    counts = jnp.sum(pc_ref[...], axis=0)
    out_ref[...] = sums / counts[None, :]


def kernel(coords, values):
    ct = coords.T
    vt = jnp.transpose(values, (0, 2, 1))
    edges = jnp.linspace(-1.0 - 1e-6, 1.0 + 1e-6, H + 1).astype(coords.dtype)
    eb = jnp.broadcast_to(edges[8:17, None], (9, 16)).astype(jnp.float32)
    psums, pcnt = _sc_partials(ct, eb, vt)
    means = pl.pallas_call(
        _combine_body,
        out_shape=jax.ShapeDtypeStruct((B * C, NBIN), jnp.float32),
    )(psums, pcnt)
    return jnp.transpose(means.reshape(B, C, NBIN), (0, 2, 1)).reshape(B, NBIN * C)
